# Initial kernel scaffold; baseline (speedup 1.0000x reference)
import jax
import jax.numpy as jnp
from jax.experimental import pallas as pl


def kernel(prediction, target, mask):
    raise NotImplementedError("write your pallas kernel here")


# speedup vs baseline: 11.1898x; 11.1898x over previous
import numpy as np

import jax
import jax.numpy as jnp
from jax import lax
from jax.experimental import pallas as pl
from jax.experimental.pallas import tpu as pltpu
from jax.experimental.pallas import tpu_sc as plsc

B, H, W = 32, 512, 512
N = B * H * W
ROWS, COLS = 8192, 1024
BLK_ROWS = 256
TC_GRID = ROWS // BLK_ROWS

NT = 16
PER_TILE = N // NT
CHUNK = 16384
NCH = PER_TILE // CHUNK
HB = 2048
PAD = 17
BIG = np.int32(2**30)


def _tc_residuals(pred_ref, targ_ref, mask_ref, res_ref, scal_ref):
    i = pl.program_id(0)

    @pl.when(i == 0)
    def _():
        scal_ref[1] = 0

    m = mask_ref[...]
    r = jnp.abs(pred_ref[...] - targ_ref[...])
    res_ref[...] = jnp.where(m, r, jnp.float32(1e30))
    scal_ref[1] += jnp.sum(m.astype(jnp.int32))

    @pl.when(i == TC_GRID - 1)
    def _():
        nv = scal_ref[1]
        scal_ref[0] = (nv * 4) // 5


def _sc_select(res_hbm, scal_hbm, out_hbm,
               buf, hist, mred, mbuf, macc, scalv, outv,
               sh_hist, sem0, sem1):
    tid = lax.axis_index("s")
    base = tid * PER_TILE
    iota16 = lax.iota(jnp.int32, 16)
    ones_i = jnp.ones((16,), jnp.int32)
    zeros_i = jnp.zeros((16,), jnp.int32)

    pltpu.sync_copy(scal_hbm, scalv)
    sv = scalv[...]
    keep = sv[0]
    num_valid = sv[1]

    def start(ci, b, sem):
        pltpu.async_copy(res_hbm.at[pl.ds(base + ci * CHUNK, CHUNK)],
                         buf.at[b], sem)

    def wait(ci, b, sem):
        pltpu.make_async_copy(res_hbm.at[pl.ds(base + ci * CHUNK, CHUNK)],
                              buf.at[b], sem).wait()

    def scan(process, carry_init):
        start(0, 0, sem0)

        def body(j, carry):
            wait(2 * j, 0, sem0)
            start(2 * j + 1, 1, sem1)
            carry = process(0, carry)
            wait(2 * j + 1, 1, sem1)

            @pl.when(j < NCH // 2 - 1)
            def _():
                start(2 * j + 2, 0, sem0)

            return process(1, carry)

        return lax.fori_loop(0, NCH // 2, body, carry_init)

    def zero_hist():
        def body(i, c):
            for u in range(8):
                hist[pl.ds((i * 8 + u) * 16, 16)] = zeros_i
            return c
        lax.fori_loop(0, HB * PAD // 128, body, 0)

    def hist_pass(bucket_fn):
        zero_hist()

        def process(b, carry):
            def body(i, c):
                for u in range(8):
                    v = buf[b, pl.ds(i * 128 + u * 16, 16)]
                    bits = plsc.bitcast(v, jnp.int32)
                    bk, match = bucket_fn(bits)
                    idx = bk * PAD + iota16
                    plsc.addupdate_scatter(hist, [idx], ones_i, mask=match)
                return c
            return lax.fori_loop(0, CHUNK // 128, body, carry)

        scan(process, 0)

        def red(i, c):
            rows = (i * 16 + iota16) * PAD
            acc = plsc.load_gather(hist, [rows])
            for l in range(1, 16):
                acc = acc + plsc.load_gather(hist, [rows + l])
            mred[pl.ds(i * 16, 16)] = acc
            return c
        lax.fori_loop(0, HB // 16, red, 0)

        pltpu.sync_copy(mred, sh_hist.at[tid])
        plsc.subcore_barrier()
        pltpu.sync_copy(sh_hist.at[0], macc)
        for u in range(1, NT):
            pltpu.sync_copy(sh_hist.at[u], mbuf)

            def add(i, c):
                for w in range(8):
                    o = (i * 8 + w) * 16
                    plsc.addupdate(macc.at[pl.ds(o, 16)], mbuf[pl.ds(o, 16)])
                return c
            lax.fori_loop(0, HB // 128, add, 0)
        plsc.subcore_barrier()

    def choose(k_res):
        def b1(i, carry):
            total, best = carry
            v = macc[pl.ds(i * 16, 16)]
            cum = plsc.cumsum(v) + total
            lanes = iota16 + i * 16
            cand = jnp.min(jnp.where(cum >= k_res, lanes, BIG))
            return jnp.max(cum), jnp.minimum(best, cand)

        _, best = lax.fori_loop(0, HB // 16, b1, (jnp.int32(0), BIG))

        def b2(i, acc):
            v = macc[pl.ds(i * 16, 16)]
            lanes = iota16 + i * 16
            return acc + jnp.sum(jnp.where(lanes < best, v, 0))

        below = lax.fori_loop(0, HB // 16, b2, jnp.int32(0))
        return best, k_res - below

    hist_pass(lambda bits: (lax.shift_right_logical(bits, 21), None))
    b1, k1 = choose(keep)

    hist_pass(lambda bits: (
        jnp.bitwise_and(lax.shift_right_logical(bits, 10), jnp.int32(0x7FF)),
        lax.shift_right_logical(bits, 21) == b1))
    b2, k2 = choose(k1)
    p2 = jnp.bitwise_or(lax.shift_left(b1, 11), b2)

    hist_pass(lambda bits: (
        jnp.bitwise_and(bits, jnp.int32(0x3FF)),
        lax.shift_right_logical(bits, 10) == p2))
    b3, _ = choose(k2)
    vbits = jnp.bitwise_or(lax.shift_left(p2, 10), b3)
    vcv = plsc.bitcast(jnp.full((16,), vbits, jnp.int32), jnp.float32)

    def process4(b, carry):
        def body(i, c):
            sumv, cntv = c
            for u in range(8):
                v = buf[b, pl.ds(i * 128 + u * 16, 16)]
                m = v < vcv
                sumv = sumv + jnp.where(m, v, jnp.float32(0.0))
                cntv = cntv + jnp.where(m, 1, 0)
            return sumv, cntv
        return lax.fori_loop(0, CHUNK // 128, body, carry)

    sumv, cntv = scan(process4,
                      (jnp.zeros((16,), jnp.float32), jnp.zeros((16,), jnp.int32)))
    sl = jnp.sum(sumv)
    cl = jnp.sum(cntv).astype(jnp.float32)
    pv = (jnp.where(iota16 == 0, sl, jnp.float32(0.0))
          + jnp.where(iota16 == 1, cl, jnp.float32(0.0)))
    mred[pl.ds(0, 16)] = plsc.bitcast(pv, jnp.int32)
    pltpu.sync_copy(mred, sh_hist.at[tid])
    plsc.subcore_barrier()

    @pl.when(tid == 0)
    def _():
        tot = jnp.zeros((16,), jnp.float32)
        for u in range(NT):
            pltpu.sync_copy(sh_hist.at[u], mbuf)
            tot = tot + plsc.bitcast(mbuf[pl.ds(0, 16)], jnp.float32)
        sum_less = jnp.full((16,), tot[0], jnp.float32)
        cnt_less = jnp.full((16,), tot[1], jnp.float32)
        kf = jnp.full((16,), keep.astype(jnp.float32), jnp.float32)
        trimmed = sum_less + (kf - cnt_less) * vcv
        nvf = jnp.full((16,), num_valid.astype(jnp.float32), jnp.float32)
        divisor = jnp.maximum(nvf, jnp.float32(1.0))
        result = jnp.where(keep > 0, trimmed / divisor,
                           jnp.zeros((16,), jnp.float32))
        outv[...] = result
        pltpu.sync_copy(outv, out_hbm)


@jax.jit
def kernel(prediction, target, mask):
    p = prediction.reshape(ROWS, COLS)
    t = target.reshape(ROWS, COLS)
    m = mask.reshape(ROWS, COLS)

    res, scal = pl.pallas_call(
        _tc_residuals,
        grid=(TC_GRID,),
        in_specs=[
            pl.BlockSpec((BLK_ROWS, COLS), lambda i: (i, 0)),
            pl.BlockSpec((BLK_ROWS, COLS), lambda i: (i, 0)),
            pl.BlockSpec((BLK_ROWS, COLS), lambda i: (i, 0)),
        ],
        out_specs=[
            pl.BlockSpec((BLK_ROWS, COLS), lambda i: (i, 0)),
            pl.BlockSpec(memory_space=pltpu.SMEM),
        ],
        out_shape=[
            jax.ShapeDtypeStruct((ROWS, COLS), jnp.float32),
            jax.ShapeDtypeStruct((16,), jnp.int32),
        ],
    )(p, t, m)

    sc = pl.kernel(
        _sc_select,
        out_type=jax.ShapeDtypeStruct((16,), jnp.float32),
        mesh=plsc.VectorSubcoreMesh(core_axis_name="c", subcore_axis_name="s",
                                    num_cores=1),
        compiler_params=pltpu.CompilerParams(needs_layout_passes=False),
        scratch_types=[
            pltpu.VMEM((2, CHUNK), jnp.float32),
            pltpu.VMEM((HB * PAD,), jnp.int32),
            pltpu.VMEM((HB,), jnp.int32),
            pltpu.VMEM((HB,), jnp.int32),
            pltpu.VMEM((HB,), jnp.int32),
            pltpu.VMEM((16,), jnp.int32),
            pltpu.VMEM((16,), jnp.float32),
            pltpu.VMEM_SHARED((NT, HB), jnp.int32),
            pltpu.SemaphoreType.DMA,
            pltpu.SemaphoreType.DMA,
        ],
    )
    out16 = sc(res.reshape(N), scal)
    return out16[0]


# speedup vs baseline: 28.4648x; 2.5438x over previous
import numpy as np

import jax
import jax.numpy as jnp
from jax import lax
from jax.experimental import pallas as pl
from jax.experimental.pallas import tpu as pltpu
from jax.experimental.pallas import tpu_sc as plsc

B, H, W = 32, 512, 512
N = B * H * W
ROWS, COLS = 8192, 1024
BLK_ROWS = 256
TC_GRID = ROWS // BLK_ROWS

NT = 16
PER_TILE = N // NT
CHUNK = 16384
NCH = PER_TILE // CHUNK
HB = 2048
PAD = 17
BIG = np.int32(2**30)


def _tc_residuals(pred_ref, targ_ref, mask_ref, res_ref, scal_ref):
    i = pl.program_id(0)

    @pl.when(i == 0)
    def _():
        scal_ref[1] = 0

    m = mask_ref[...]
    r = jnp.abs(pred_ref[...] - targ_ref[...])
    res_ref[...] = jnp.where(m, r, jnp.float32(1e30))
    scal_ref[1] += jnp.sum(m.astype(jnp.int32))

    @pl.when(i == TC_GRID - 1)
    def _():
        nv = scal_ref[1]
        scal_ref[0] = (nv * 4) // 5


def _sc_select(res_hbm, scal_hbm, out_hbm,
               buf, hist, mred, mgbuf, macc, scalv, outv,
               sh_hist, sem0, sem1):
    tid = lax.axis_index("s")
    base = tid * PER_TILE
    iota16 = lax.iota(jnp.int32, 16)
    ones_i = jnp.ones((16,), jnp.int32)
    zeros_i = jnp.zeros((16,), jnp.int32)

    pltpu.sync_copy(scal_hbm, scalv)
    sv = scalv[...]
    keep = sv[0]
    num_valid = sv[1]

    def start(ci, b, sem):
        pltpu.async_copy(res_hbm.at[pl.ds(base + ci * CHUNK, CHUNK)],
                         buf.at[b], sem)

    def wait(ci, b, sem):
        pltpu.make_async_copy(res_hbm.at[pl.ds(base + ci * CHUNK, CHUNK)],
                              buf.at[b], sem).wait()

    def scan(process, carry_init):
        start(0, 0, sem0)

        def body(j, carry):
            wait(2 * j, 0, sem0)
            start(2 * j + 1, 1, sem1)
            carry = process(0, carry)
            wait(2 * j + 1, 1, sem1)

            @pl.when(j < NCH // 2 - 1)
            def _():
                start(2 * j + 2, 0, sem0)

            return process(1, carry)

        return lax.fori_loop(0, NCH // 2, body, carry_init)

    def zero_hist():
        @plsc.parallel_loop(0, HB * PAD // 16, unroll=8)
        def _(i):
            hist[pl.ds(i * 16, 16)] = zeros_i

    def hist_pass(bucket_fn):
        zero_hist()

        def process(b, carry):
            @plsc.parallel_loop(0, CHUNK // 16, unroll=8)
            def _(i):
                v = buf[b, pl.ds(i * 16, 16)]
                bits = plsc.bitcast(v, jnp.int32)
                bk, match = bucket_fn(bits)
                idx = bk * PAD + iota16
                plsc.addupdate_scatter(hist, [idx], ones_i, mask=match)
            return carry

        scan(process, 0)

        @plsc.parallel_loop(0, HB // 16, unroll=2)
        def _(i):
            rows = (i * 16 + iota16) * PAD
            acc = plsc.load_gather(hist, [rows])
            for l in range(1, 16):
                acc = acc + plsc.load_gather(hist, [rows + l])
            mred[pl.ds(i * 16, 16)] = acc

        pltpu.sync_copy(mred, sh_hist.at[tid])
        plsc.subcore_barrier()
        pltpu.sync_copy(sh_hist, mgbuf)

        @plsc.parallel_loop(0, HB // 16, unroll=2)
        def _(i):
            acc = mgbuf[0, pl.ds(i * 16, 16)]
            for u in range(1, NT):
                acc = acc + mgbuf[u, pl.ds(i * 16, 16)]
            macc[pl.ds(i * 16, 16)] = acc
        plsc.subcore_barrier()

    def choose(k_res):
        def b1(i, carry):
            total, best = carry
            v = macc[pl.ds(i * 16, 16)]
            cum = plsc.cumsum(v) + total
            lanes = iota16 + i * 16
            cand = jnp.min(jnp.where(cum >= k_res, lanes, BIG))
            return jnp.max(cum), jnp.minimum(best, cand)

        _, best = lax.fori_loop(0, HB // 16, b1, (jnp.int32(0), BIG))

        def b2(i, acc):
            v = macc[pl.ds(i * 16, 16)]
            lanes = iota16 + i * 16
            return acc + jnp.sum(jnp.where(lanes < best, v, 0))

        below = lax.fori_loop(0, HB // 16, b2, jnp.int32(0))
        return best, k_res - below

    hist_pass(lambda bits: (lax.shift_right_logical(bits, 21), None))
    b1, k1 = choose(keep)

    hist_pass(lambda bits: (
        jnp.bitwise_and(lax.shift_right_logical(bits, 10), jnp.int32(0x7FF)),
        lax.shift_right_logical(bits, 21) == b1))
    b2, k2 = choose(k1)
    p2 = jnp.bitwise_or(lax.shift_left(b1, 11), b2)

    hist_pass(lambda bits: (
        jnp.bitwise_and(bits, jnp.int32(0x3FF)),
        lax.shift_right_logical(bits, 10) == p2))
    b3, _ = choose(k2)
    vbits = jnp.bitwise_or(lax.shift_left(p2, 10), b3)
    vcv = plsc.bitcast(jnp.full((16,), vbits, jnp.int32), jnp.float32)

    def process4(b, carry):
        @plsc.parallel_loop(0, CHUNK // 16, 8, carry=carry)
        def out(i, c):
            sumv, cntv = c
            sv, cv = [], []
            for u in range(8):
                v = buf[b, pl.ds((i + u) * 16, 16)]
                m = v < vcv
                sv.append(jnp.where(m, v, jnp.float32(0.0)))
                cv.append(jnp.where(m, 1, 0))
            while len(sv) > 1:
                sv = [a + b2 for a, b2 in zip(sv[::2], sv[1::2])]
                cv = [a + b2 for a, b2 in zip(cv[::2], cv[1::2])]
            return sumv + sv[0], cntv + cv[0]
        return out

    sumv, cntv = scan(process4,
                      (jnp.zeros((16,), jnp.float32), jnp.zeros((16,), jnp.int32)))
    sl = jnp.sum(sumv)
    cl = jnp.sum(cntv).astype(jnp.float32)
    pv = (jnp.where(iota16 == 0, sl, jnp.float32(0.0))
          + jnp.where(iota16 == 1, cl, jnp.float32(0.0)))
    mred[pl.ds(0, 16)] = plsc.bitcast(pv, jnp.int32)
    pltpu.sync_copy(mred, sh_hist.at[tid])
    plsc.subcore_barrier()

    @pl.when(tid == 0)
    def _():
        pltpu.sync_copy(sh_hist, mgbuf)
        tot = jnp.zeros((16,), jnp.float32)
        for u in range(NT):
            tot = tot + plsc.bitcast(mgbuf[u, pl.ds(0, 16)], jnp.float32)
        sum_less = jnp.full((16,), tot[0], jnp.float32)
        cnt_less = jnp.full((16,), tot[1], jnp.float32)
        kf = jnp.full((16,), keep.astype(jnp.float32), jnp.float32)
        trimmed = sum_less + (kf - cnt_less) * vcv
        nvf = jnp.full((16,), num_valid.astype(jnp.float32), jnp.float32)
        divisor = jnp.maximum(nvf, jnp.float32(1.0))
        result = jnp.where(keep > 0, trimmed / divisor,
                           jnp.zeros((16,), jnp.float32))
        outv[...] = result
        pltpu.sync_copy(outv, out_hbm)


@jax.jit
def kernel(prediction, target, mask):
    p = prediction.reshape(ROWS, COLS)
    t = target.reshape(ROWS, COLS)
    m = mask.reshape(ROWS, COLS)

    res, scal = pl.pallas_call(
        _tc_residuals,
        grid=(TC_GRID,),
        in_specs=[
            pl.BlockSpec((BLK_ROWS, COLS), lambda i: (i, 0)),
            pl.BlockSpec((BLK_ROWS, COLS), lambda i: (i, 0)),
            pl.BlockSpec((BLK_ROWS, COLS), lambda i: (i, 0)),
        ],
        out_specs=[
            pl.BlockSpec((BLK_ROWS, COLS), lambda i: (i, 0)),
            pl.BlockSpec(memory_space=pltpu.SMEM),
        ],
        out_shape=[
            jax.ShapeDtypeStruct((ROWS, COLS), jnp.float32),
            jax.ShapeDtypeStruct((16,), jnp.int32),
        ],
    )(p, t, m)

    sc = pl.kernel(
        _sc_select,
        out_type=jax.ShapeDtypeStruct((16,), jnp.float32),
        mesh=plsc.VectorSubcoreMesh(core_axis_name="c", subcore_axis_name="s",
                                    num_cores=1),
        compiler_params=pltpu.CompilerParams(needs_layout_passes=False),
        scratch_types=[
            pltpu.VMEM((2, CHUNK), jnp.float32),
            pltpu.VMEM((HB * PAD,), jnp.int32),
            pltpu.VMEM((HB,), jnp.int32),
            pltpu.VMEM((NT, HB), jnp.int32),
            pltpu.VMEM((HB,), jnp.int32),
            pltpu.VMEM((16,), jnp.int32),
            pltpu.VMEM((16,), jnp.float32),
            pltpu.VMEM_SHARED((NT, HB), jnp.int32),
            pltpu.SemaphoreType.DMA,
            pltpu.SemaphoreType.DMA,
        ],
    )
    out16 = sc(res.reshape(N), scal)
    return out16[0]


# speedup vs baseline: 34.1166x; 1.1986x over previous
import numpy as np

import jax
import jax.numpy as jnp
from jax import lax
from jax.experimental import pallas as pl
from jax.experimental.pallas import tpu as pltpu
from jax.experimental.pallas import tpu_sc as plsc

B, H, W = 32, 512, 512
N = B * H * W
ROWS, COLS = 16384, 512

NT = 16
PER_TILE = N // NT
TROWS = ROWS // NT
CH = 8192
CROWS = CH // COLS
NCH = PER_TILE // CH
HB = 2048
PAD = 17
BIG = np.int32(2**30)


def _sc_select(pred_hbm, targ_hbm, maskf_hbm, out_hbm, res_hbm,
               bufp, buft, bufm, bufo, hist, mred, mgbuf, macc, outv,
               sh_hist, semi0, semi1, semo0, semo1):
    tid = lax.axis_index("s")
    base = tid * PER_TILE
    rbase = tid * TROWS
    iota16 = lax.iota(jnp.int32, 16)
    ones_i = jnp.ones((16,), jnp.int32)
    zeros_i = jnp.zeros((16,), jnp.int32)
    semi = (semi0, semi1)
    semo = (semo0, semo1)
    inbufs = (bufp, buft, bufm)

    @plsc.parallel_loop(0, HB * PAD // 16, unroll=8)
    def _(i):
        hist[pl.ds(i * 16, 16)] = zeros_i

    def start3(c, par):
        r0 = rbase + c * CROWS
        for hb, vb in ((pred_hbm, bufp), (targ_hbm, buft), (maskf_hbm, bufm)):
            pltpu.async_copy(hb.at[pl.ds(r0, CROWS)],
                             vb.at[pl.ds(par * CROWS, CROWS)], semi[par])

    def wait3(c, par):
        r0 = rbase + c * CROWS
        for hb, vb in ((pred_hbm, bufp), (targ_hbm, buft), (maskf_hbm, bufm)):
            pltpu.make_async_copy(hb.at[pl.ds(r0, CROWS)],
                                  vb.at[pl.ds(par * CROWS, CROWS)],
                                  semi[par]).wait()

    def p1_compute(c, par, cnt):
        @plsc.parallel_loop(0, CH // 16, carry=cnt)
        def out(i, cv):
            rr = par * CROWS + lax.shift_right_logical(i, 5)
            cc = lax.shift_left(jnp.bitwise_and(i, jnp.int32(31)), 4)
            p = bufp[rr, pl.ds(cc, 16)]
            t = buft[rr, pl.ds(cc, 16)]
            m = bufm[rr, pl.ds(cc, 16)]
            r = jnp.where(m > jnp.float32(0.5), jnp.abs(p - t),
                          jnp.float32(1e30))
            bufo[par, pl.ds(i * 16, 16)] = r
            bits = plsc.bitcast(r, jnp.int32)
            idx = lax.shift_right_logical(bits, 21) * PAD + iota16
            plsc.addupdate_scatter(hist, [idx], ones_i)
            return cv + jnp.where(m > jnp.float32(0.5), 1, 0)
        return out

    start3(0, 0)
    cnt0 = jnp.zeros((16,), jnp.int32)

    def p1_body(j, cnt):
        c0 = 2 * j
        wait3(c0, 0)
        start3(c0 + 1, 1)
        cnt = p1_compute(c0, 0, cnt)

        @pl.when(j >= 1)
        def _():
            pltpu.make_async_copy(bufo.at[0], res_hbm.at[pl.ds(0, CH)],
                                  semo[0]).wait()
        pltpu.async_copy(bufo.at[0], res_hbm.at[pl.ds(base + c0 * CH, CH)],
                         semo[0])

        wait3(c0 + 1, 1)

        @pl.when(j < NCH // 2 - 1)
        def _():
            start3(c0 + 2, 0)
        cnt = p1_compute(c0 + 1, 1, cnt)

        @pl.when(j >= 1)
        def _():
            pltpu.make_async_copy(bufo.at[1], res_hbm.at[pl.ds(0, CH)],
                                  semo[1]).wait()
        pltpu.async_copy(bufo.at[1], res_hbm.at[pl.ds(base + (c0 + 1) * CH, CH)],
                         semo[1])
        return cnt

    cnt = lax.fori_loop(0, NCH // 2, p1_body, cnt0)
    pltpu.make_async_copy(bufo.at[0], res_hbm.at[pl.ds(0, CH)], semo[0]).wait()
    pltpu.make_async_copy(bufo.at[1], res_hbm.at[pl.ds(0, CH)], semo[1]).wait()

    def lane_reduce():
        @plsc.parallel_loop(0, HB // 16, unroll=2)
        def _(i):
            rows = (i * 16 + iota16) * PAD
            acc = plsc.load_gather(hist, [rows])
            for l in range(1, 16):
                acc = acc + plsc.load_gather(hist, [rows + l])
            mred[pl.ds(i * 16, 16)] = acc

    def merge():
        pltpu.sync_copy(mred, sh_hist.at[tid])
        plsc.subcore_barrier()
        for half in range(2):
            pltpu.sync_copy(sh_hist.at[pl.ds(half * 8, 8)], mgbuf)

            @plsc.parallel_loop(0, HB // 16, unroll=2)
            def _(i):
                acc = mgbuf[0, pl.ds(i * 16, 16)]
                for u in range(1, 8):
                    acc = acc + mgbuf[u, pl.ds(i * 16, 16)]
                if half == 0:
                    macc[pl.ds(i * 16, 16)] = acc
                else:
                    plsc.addupdate(macc.at[pl.ds(i * 16, 16)], acc)
        plsc.subcore_barrier()

    lane_reduce()
    cnt_s = jnp.sum(cnt)
    mred[pl.ds(1024, 16)] = jnp.where(iota16 == 0, cnt_s, 0)
    merge()

    nvv = macc[pl.ds(1024, 16)]
    num_valid = nvv[0]
    nv4 = num_valid * 4
    q0 = (num_valid.astype(jnp.float32) * jnp.float32(0.8)).astype(jnp.int32)
    keep = jnp.int32(0)
    for d in range(-2, 3):
        cand = q0 + d
        ok = jnp.logical_and(cand >= 0, cand * 5 <= nv4)
        keep = jnp.where(ok, jnp.maximum(keep, cand), keep)

    def choose(k_res):
        def b1(i, carry):
            total, best = carry
            v = macc[pl.ds(i * 16, 16)]
            cum = plsc.cumsum(v) + total
            lanes = iota16 + i * 16
            cand = jnp.min(jnp.where(cum >= k_res, lanes, BIG))
            return jnp.max(cum), jnp.minimum(best, cand)

        _, best = lax.fori_loop(0, HB // 16, b1, (jnp.int32(0), BIG))

        def b2(i, acc):
            v = macc[pl.ds(i * 16, 16)]
            lanes = iota16 + i * 16
            return acc + jnp.sum(jnp.where(lanes < best, v, 0))

        below = lax.fori_loop(0, HB // 16, b2, jnp.int32(0))
        return best, k_res - below

    b1sel, k1 = choose(keep)

    def start_r(c, par):
        pltpu.async_copy(res_hbm.at[pl.ds(base + c * CH, CH)],
                         bufo.at[par], semi[par])

    def wait_r(c, par):
        pltpu.make_async_copy(res_hbm.at[pl.ds(base + c * CH, CH)],
                              bufo.at[par], semi[par]).wait()

    def scan(process, carry_init):
        start_r(0, 0)

        def body(j, carry):
            wait_r(2 * j, 0)
            start_r(2 * j + 1, 1)
            carry = process(0, carry)
            wait_r(2 * j + 1, 1)

            @pl.when(j < NCH // 2 - 1)
            def _():
                start_r(2 * j + 2, 0)

            return process(1, carry)

        return lax.fori_loop(0, NCH // 2, body, carry_init)

    def hist_pass(bucket_fn):
        @plsc.parallel_loop(0, HB * PAD // 16, unroll=8)
        def _(i):
            hist[pl.ds(i * 16, 16)] = zeros_i

        def process(b, carry):
            @plsc.parallel_loop(0, CH // 16, unroll=8)
            def _(i):
                v = bufo[b, pl.ds(i * 16, 16)]
                bits = plsc.bitcast(v, jnp.int32)
                bk, match = bucket_fn(bits)
                idx = bk * PAD + iota16
                plsc.addupdate_scatter(hist, [idx], ones_i, mask=match)
            return carry

        scan(process, 0)
        lane_reduce()
        merge()

    hist_pass(lambda bits: (
        jnp.bitwise_and(lax.shift_right_logical(bits, 10), jnp.int32(0x7FF)),
        lax.shift_right_logical(bits, 21) == b1sel))
    b2sel, k2 = choose(k1)
    p2 = jnp.bitwise_or(lax.shift_left(b1sel, 11), b2sel)

    hist_pass(lambda bits: (
        jnp.bitwise_and(bits, jnp.int32(0x3FF)),
        lax.shift_right_logical(bits, 10) == p2))
    b3sel, _ = choose(k2)
    vbits = jnp.bitwise_or(lax.shift_left(p2, 10), b3sel)
    vcv = plsc.bitcast(jnp.full((16,), vbits, jnp.int32), jnp.float32)

    def process4(b, carry):
        @plsc.parallel_loop(0, CH // 16, 8, carry=carry)
        def out(i, c):
            sumv, cntv = c
            sv, cv = [], []
            for u in range(8):
                v = bufo[b, pl.ds((i + u) * 16, 16)]
                m = v < vcv
                sv.append(jnp.where(m, v, jnp.float32(0.0)))
                cv.append(jnp.where(m, 1, 0))
            while len(sv) > 1:
                sv = [a + b2 for a, b2 in zip(sv[::2], sv[1::2])]
                cv = [a + b2 for a, b2 in zip(cv[::2], cv[1::2])]
            return sumv + sv[0], cntv + cv[0]
        return out

    sumv, cntv = scan(process4,
                      (jnp.zeros((16,), jnp.float32), jnp.zeros((16,), jnp.int32)))

    sl = jnp.sum(sumv)
    cl = jnp.sum(cntv).astype(jnp.float32)
    pv = (jnp.where(iota16 == 0, sl, jnp.float32(0.0))
          + jnp.where(iota16 == 1, cl, jnp.float32(0.0)))
    mred[pl.ds(0, 16)] = plsc.bitcast(pv, jnp.int32)
    pltpu.sync_copy(mred, sh_hist.at[tid])
    plsc.subcore_barrier()

    @pl.when(tid == 0)
    def _():
        pltpu.sync_copy(sh_hist.at[pl.ds(0, 8)], mgbuf)
        tot = jnp.zeros((16,), jnp.float32)
        for u in range(8):
            tot = tot + plsc.bitcast(mgbuf[u, pl.ds(0, 16)], jnp.float32)
        pltpu.sync_copy(sh_hist.at[pl.ds(8, 8)], mgbuf)
        for u in range(8):
            tot = tot + plsc.bitcast(mgbuf[u, pl.ds(0, 16)], jnp.float32)
        sum_less = jnp.full((16,), tot[0], jnp.float32)
        cnt_less = jnp.full((16,), tot[1], jnp.float32)
        kf = jnp.full((16,), keep.astype(jnp.float32), jnp.float32)
        trimmed = sum_less + (kf - cnt_less) * vcv
        nvf = jnp.full((16,), num_valid.astype(jnp.float32), jnp.float32)
        divisor = jnp.maximum(nvf, jnp.float32(1.0))
        result = jnp.where(keep > 0, trimmed / divisor,
                           jnp.zeros((16,), jnp.float32))
        outv[...] = result
        pltpu.sync_copy(outv, out_hbm)


@jax.jit
def kernel(prediction, target, mask):
    p = prediction.reshape(ROWS, COLS)
    t = target.reshape(ROWS, COLS)
    mf = mask.astype(jnp.float32).reshape(ROWS, COLS)

    sc = pl.kernel(
        _sc_select,
        out_type=(jax.ShapeDtypeStruct((16,), jnp.float32),
                  jax.ShapeDtypeStruct((N,), jnp.float32)),
        mesh=plsc.VectorSubcoreMesh(core_axis_name="c", subcore_axis_name="s",
                                    num_cores=1),
        compiler_params=pltpu.CompilerParams(needs_layout_passes=False),
        scratch_types=[
            pltpu.VMEM((2 * CROWS, COLS), jnp.float32),
            pltpu.VMEM((2 * CROWS, COLS), jnp.float32),
            pltpu.VMEM((2 * CROWS, COLS), jnp.float32),
            pltpu.VMEM((2, CH), jnp.float32),
            pltpu.VMEM((HB * PAD,), jnp.int32),
            pltpu.VMEM((HB,), jnp.int32),
            pltpu.VMEM((8, HB), jnp.int32),
            pltpu.VMEM((HB,), jnp.int32),
            pltpu.VMEM((16,), jnp.float32),
            pltpu.VMEM_SHARED((NT, HB), jnp.int32),
            pltpu.SemaphoreType.DMA,
            pltpu.SemaphoreType.DMA,
            pltpu.SemaphoreType.DMA,
            pltpu.SemaphoreType.DMA,
        ],
    )
    out16, _ = sc(p, t, mf)
    return out16[0]


# speedup vs baseline: 38.3023x; 1.1227x over previous
import numpy as np

import jax
import jax.numpy as jnp
from jax import lax
from jax.experimental import pallas as pl
from jax.experimental.pallas import tpu as pltpu
from jax.experimental.pallas import tpu_sc as plsc

B, H, W = 32, 512, 512
N = B * H * W
ROWS, COLS = 16384, 512

NT = 16
PER_TILE = N // NT
TROWS = ROWS // NT
CH = 8192
CROWS = CH // COLS
NCH = PER_TILE // CH
HB = 2048
PAD = 17
BIG = np.int32(2**30)


def _sc_select(pred_hbm, targ_hbm, maskf_hbm, out_hbm, res_hbm,
               bufp, buft, bufm, bufo, hist, mred, mgbuf, macc, outv,
               sh_hist, semi0, semi1, semo0, semo1):
    tid = lax.axis_index("s")
    base = tid * PER_TILE
    rbase = tid * TROWS
    iota16 = lax.iota(jnp.int32, 16)
    ones_i = jnp.ones((16,), jnp.int32)
    zeros_i = jnp.zeros((16,), jnp.int32)
    semi = (semi0, semi1)
    semo = (semo0, semo1)
    inbufs = (bufp, buft, bufm)

    @plsc.parallel_loop(0, HB * PAD // 16, unroll=8)
    def _(i):
        hist[pl.ds(i * 16, 16)] = zeros_i

    def start3(c, par):
        r0 = rbase + c * CROWS
        for hb, vb in ((pred_hbm, bufp), (targ_hbm, buft), (maskf_hbm, bufm)):
            pltpu.async_copy(hb.at[pl.ds(r0, CROWS)],
                             vb.at[pl.ds(par * CROWS, CROWS)], semi[par])

    def wait3(c, par):
        r0 = rbase + c * CROWS
        for hb, vb in ((pred_hbm, bufp), (targ_hbm, buft), (maskf_hbm, bufm)):
            pltpu.make_async_copy(hb.at[pl.ds(r0, CROWS)],
                                  vb.at[pl.ds(par * CROWS, CROWS)],
                                  semi[par]).wait()

    def p1_compute(c, par, cnt):
        @plsc.parallel_loop(0, CH // 16, 4, carry=cnt)
        def out(i, cv):
            cs = []
            for u in range(4):
                iu = i + u
                rr = par * CROWS + lax.shift_right_logical(iu, 5)
                cc = lax.shift_left(jnp.bitwise_and(iu, jnp.int32(31)), 4)
                p = bufp[rr, pl.ds(cc, 16)]
                t = buft[rr, pl.ds(cc, 16)]
                m = bufm[rr, pl.ds(cc, 16)]
                valid = m > jnp.float32(0.5)
                r = jnp.where(valid, jnp.abs(p - t), jnp.float32(1e30))
                bufo[par, pl.ds(iu * 16, 16)] = r
                bits = plsc.bitcast(r, jnp.int32)
                idx = lax.shift_right_logical(bits, 21) * PAD + iota16
                plsc.addupdate_scatter(hist, [idx], ones_i)
                cs.append(jnp.where(valid, 1, 0))
            return cv + ((cs[0] + cs[1]) + (cs[2] + cs[3]))
        return out

    start3(0, 0)
    cnt0 = jnp.zeros((16,), jnp.int32)

    def p1_body(j, cnt):
        c0 = 2 * j
        wait3(c0, 0)
        start3(c0 + 1, 1)
        cnt = p1_compute(c0, 0, cnt)

        @pl.when(j >= 1)
        def _():
            pltpu.make_async_copy(bufo.at[0], res_hbm.at[pl.ds(0, CH)],
                                  semo[0]).wait()
        pltpu.async_copy(bufo.at[0], res_hbm.at[pl.ds(base + c0 * CH, CH)],
                         semo[0])

        wait3(c0 + 1, 1)

        @pl.when(j < NCH // 2 - 1)
        def _():
            start3(c0 + 2, 0)
        cnt = p1_compute(c0 + 1, 1, cnt)

        @pl.when(j >= 1)
        def _():
            pltpu.make_async_copy(bufo.at[1], res_hbm.at[pl.ds(0, CH)],
                                  semo[1]).wait()
        pltpu.async_copy(bufo.at[1], res_hbm.at[pl.ds(base + (c0 + 1) * CH, CH)],
                         semo[1])
        return cnt

    cnt = lax.fori_loop(0, NCH // 2, p1_body, cnt0)
    pltpu.make_async_copy(bufo.at[0], res_hbm.at[pl.ds(0, CH)], semo[0]).wait()
    pltpu.make_async_copy(bufo.at[1], res_hbm.at[pl.ds(0, CH)], semo[1]).wait()

    def lane_reduce():
        @plsc.parallel_loop(0, HB // 16, unroll=2)
        def _(i):
            rows = (i * 16 + iota16) * PAD
            acc = plsc.load_gather(hist, [rows])
            for l in range(1, 16):
                acc = acc + plsc.load_gather(hist, [rows + l])
            mred[pl.ds(i * 16, 16)] = acc

    def merge():
        pltpu.sync_copy(mred, sh_hist.at[tid])
        plsc.subcore_barrier()
        for half in range(2):
            pltpu.sync_copy(sh_hist.at[pl.ds(half * 8, 8)], mgbuf)

            @plsc.parallel_loop(0, HB // 16, unroll=2)
            def _(i):
                acc = mgbuf[0, pl.ds(i * 16, 16)]
                for u in range(1, 8):
                    acc = acc + mgbuf[u, pl.ds(i * 16, 16)]
                if half == 0:
                    macc[pl.ds(i * 16, 16)] = acc
                else:
                    plsc.addupdate(macc.at[pl.ds(i * 16, 16)], acc)
        plsc.subcore_barrier()

    lane_reduce()
    cnt_s = jnp.sum(cnt)
    mred[pl.ds(1024, 16)] = jnp.where(iota16 == 0, cnt_s, 0)
    merge()

    nvv = macc[pl.ds(1024, 16)]
    num_valid = nvv[0]
    nv4 = num_valid * 4
    q0 = (num_valid.astype(jnp.float32) * jnp.float32(0.8)).astype(jnp.int32)
    keep = jnp.int32(0)
    for d in range(-2, 3):
        cand = q0 + d
        ok = jnp.logical_and(cand >= 0, cand * 5 <= nv4)
        keep = jnp.where(ok, jnp.maximum(keep, cand), keep)

    def choose(k_res):
        def b1(i, carry):
            total, best = carry
            v = macc[pl.ds(i * 16, 16)]
            cum = plsc.cumsum(v) + total
            lanes = iota16 + i * 16
            cand = jnp.min(jnp.where(cum >= k_res, lanes, BIG))
            return jnp.max(cum), jnp.minimum(best, cand)

        _, best = lax.fori_loop(0, HB // 16, b1, (jnp.int32(0), BIG))

        def b2(i, acc):
            v = macc[pl.ds(i * 16, 16)]
            lanes = iota16 + i * 16
            return acc + jnp.sum(jnp.where(lanes < best, v, 0))

        below = lax.fori_loop(0, HB // 16, b2, jnp.int32(0))
        return best, k_res - below

    b1sel, k1 = choose(keep)

    def start_r(c, par):
        pltpu.async_copy(res_hbm.at[pl.ds(base + c * CH, CH)],
                         bufo.at[par], semi[par])

    def wait_r(c, par):
        pltpu.make_async_copy(res_hbm.at[pl.ds(base + c * CH, CH)],
                              bufo.at[par], semi[par]).wait()

    def scan(process, carry_init):
        start_r(0, 0)

        def body(j, carry):
            wait_r(2 * j, 0)
            start_r(2 * j + 1, 1)
            carry = process(0, carry)
            wait_r(2 * j + 1, 1)

            @pl.when(j < NCH // 2 - 1)
            def _():
                start_r(2 * j + 2, 0)

            return process(1, carry)

        return lax.fori_loop(0, NCH // 2, body, carry_init)

    def hist_pass(bucket_fn):
        @plsc.parallel_loop(0, HB * PAD // 16, unroll=8)
        def _(i):
            hist[pl.ds(i * 16, 16)] = zeros_i

        def process(b, carry):
            @plsc.parallel_loop(0, CH // 16, unroll=8)
            def _(i):
                v = bufo[b, pl.ds(i * 16, 16)]
                bits = plsc.bitcast(v, jnp.int32)
                bk, match = bucket_fn(bits)
                idx = bk * PAD + iota16
                plsc.addupdate_scatter(hist, [idx], ones_i, mask=match)
            return carry

        scan(process, 0)
        lane_reduce()
        merge()

    hist_pass(lambda bits: (
        jnp.bitwise_and(lax.shift_right_logical(bits, 10), jnp.int32(0x7FF)),
        lax.shift_right_logical(bits, 21) == b1sel))
    b2sel, k2 = choose(k1)
    p2 = jnp.bitwise_or(lax.shift_left(b1sel, 11), b2sel)

    hist_pass(lambda bits: (
        jnp.bitwise_and(bits, jnp.int32(0x3FF)),
        lax.shift_right_logical(bits, 10) == p2))
    b3sel, _ = choose(k2)
    vbits = jnp.bitwise_or(lax.shift_left(p2, 10), b3sel)
    vcv = plsc.bitcast(jnp.full((16,), vbits, jnp.int32), jnp.float32)

    def process4(b, carry):
        @plsc.parallel_loop(0, CH // 16, 8, carry=carry)
        def out(i, c):
            sumv, cntv = c
            sv, cv = [], []
            for u in range(8):
                v = bufo[b, pl.ds((i + u) * 16, 16)]
                m = v < vcv
                sv.append(jnp.where(m, v, jnp.float32(0.0)))
                cv.append(jnp.where(m, 1, 0))
            while len(sv) > 1:
                sv = [a + b2 for a, b2 in zip(sv[::2], sv[1::2])]
                cv = [a + b2 for a, b2 in zip(cv[::2], cv[1::2])]
            return sumv + sv[0], cntv + cv[0]
        return out

    sumv, cntv = scan(process4,
                      (jnp.zeros((16,), jnp.float32), jnp.zeros((16,), jnp.int32)))

    sl = jnp.sum(sumv)
    cl = jnp.sum(cntv).astype(jnp.float32)
    pv = (jnp.where(iota16 == 0, sl, jnp.float32(0.0))
          + jnp.where(iota16 == 1, cl, jnp.float32(0.0)))
    mred[pl.ds(0, 16)] = plsc.bitcast(pv, jnp.int32)
    pltpu.sync_copy(mred, sh_hist.at[tid])
    plsc.subcore_barrier()

    @pl.when(tid == 0)
    def _():
        pltpu.sync_copy(sh_hist.at[pl.ds(0, 8)], mgbuf)
        tot = jnp.zeros((16,), jnp.float32)
        for u in range(8):
            tot = tot + plsc.bitcast(mgbuf[u, pl.ds(0, 16)], jnp.float32)
        pltpu.sync_copy(sh_hist.at[pl.ds(8, 8)], mgbuf)
        for u in range(8):
            tot = tot + plsc.bitcast(mgbuf[u, pl.ds(0, 16)], jnp.float32)
        sum_less = jnp.full((16,), tot[0], jnp.float32)
        cnt_less = jnp.full((16,), tot[1], jnp.float32)
        kf = jnp.full((16,), keep.astype(jnp.float32), jnp.float32)
        trimmed = sum_less + (kf - cnt_less) * vcv
        nvf = jnp.full((16,), num_valid.astype(jnp.float32), jnp.float32)
        divisor = jnp.maximum(nvf, jnp.float32(1.0))
        result = jnp.where(keep > 0, trimmed / divisor,
                           jnp.zeros((16,), jnp.float32))
        outv[...] = result
        pltpu.sync_copy(outv, out_hbm)


@jax.jit
def kernel(prediction, target, mask):
    p = prediction.reshape(ROWS, COLS)
    t = target.reshape(ROWS, COLS)
    mf = mask.astype(jnp.float32).reshape(ROWS, COLS)

    sc = pl.kernel(
        _sc_select,
        out_type=(jax.ShapeDtypeStruct((16,), jnp.float32),
                  jax.ShapeDtypeStruct((N,), jnp.float32)),
        mesh=plsc.VectorSubcoreMesh(core_axis_name="c", subcore_axis_name="s",
                                    num_cores=1),
        compiler_params=pltpu.CompilerParams(needs_layout_passes=False),
        scratch_types=[
            pltpu.VMEM((2 * CROWS, COLS), jnp.float32),
            pltpu.VMEM((2 * CROWS, COLS), jnp.float32),
            pltpu.VMEM((2 * CROWS, COLS), jnp.float32),
            pltpu.VMEM((2, CH), jnp.float32),
            pltpu.VMEM((HB * PAD,), jnp.int32),
            pltpu.VMEM((HB,), jnp.int32),
            pltpu.VMEM((8, HB), jnp.int32),
            pltpu.VMEM((HB,), jnp.int32),
            pltpu.VMEM((16,), jnp.float32),
            pltpu.VMEM_SHARED((NT, HB), jnp.int32),
            pltpu.SemaphoreType.DMA,
            pltpu.SemaphoreType.DMA,
            pltpu.SemaphoreType.DMA,
            pltpu.SemaphoreType.DMA,
        ],
    )
    out16, _ = sc(p, t, mf)
    return out16[0]
